# baseline (speedup 1.0000x reference)
import functools

import jax
import jax.numpy as jnp
from jax import lax
from jax.experimental import pallas as pl
from jax.experimental.pallas import tpu as pltpu
from jax.experimental.pallas import tpu_sc as plsc

_TOKENS = 16384
_D = 2048
_E = 64
_T = 1024
_NBUF = 4
_NCHUNK = _TOKENS // _T

_NC = 2
_NS = 16
_TPW = _TOKENS // (_NC * _NS)
_L = 16


def _logits_body(x_hbm, w_ref, b_ref, lgt_ref, xbuf, sems):
    i = pl.program_id(0)

    def chunk_copy(j, slot):
        return pltpu.make_async_copy(
            x_hbm.at[pl.ds(j * _T, _T), :], xbuf.at[slot], sems.at[slot])

    @pl.when(i == 0)
    def _prime():
        for s in range(_NBUF - 1):
            chunk_copy(s, s).start()

    pref = i + _NBUF - 1

    @pl.when(pref < _NCHUNK)
    def _prefetch():
        chunk_copy(pref, lax.rem(pref, _NBUF)).start()

    slot = lax.rem(i, _NBUF)
    chunk_copy(i, slot).wait()

    logits = jax.lax.dot_general(
        xbuf[slot], w_ref[...], (((1,), (1,)), ((), ())),
        preferred_element_type=jnp.float32) + b_ref[...]
    lgt_ref[...] = logits.T


def _tc_logits_t(x, W, b):
    return pl.pallas_call(
        _logits_body,
        grid=(_NCHUNK,),
        in_specs=[
            pl.BlockSpec(memory_space=pl.ANY),
            pl.BlockSpec((_E, _D), lambda i: (0, 0)),
            pl.BlockSpec((1, _E), lambda i: (0, 0)),
        ],
        out_specs=pl.BlockSpec((_E, _T), lambda i: (0, i)),
        out_shape=jax.ShapeDtypeStruct((_E, _TOKENS), jnp.float32),
        scratch_shapes=[
            pltpu.VMEM((_NBUF, _T, _D), jnp.float32),
            pltpu.SemaphoreType.DMA((_NBUF,)),
        ],
    )(x, W, b.reshape(1, _E))


def _sc_top2_body(lgt_hbm, tw_hbm, ti_hbm, buf, tws, tis):
    wid = lax.axis_index("s") * _NC + lax.axis_index("c")
    base = wid * _TPW
    pltpu.sync_copy(lgt_hbm.at[:, pl.ds(base, _TPW)], buf)

    def group(g, carry):
        t0 = g * _L
        m = buf[0, pl.ds(t0, _L)]
        for e in range(1, _E):
            m = jnp.maximum(m, buf[e, pl.ds(t0, _L)])
        s = jnp.zeros((_L,), jnp.float32)
        v1 = jnp.full((_L,), -1.0, jnp.float32)
        i1 = jnp.zeros((_L,), jnp.int32)
        v2 = jnp.full((_L,), -1.0, jnp.float32)
        i2 = jnp.zeros((_L,), jnp.int32)
        for e in range(_E):
            ecol = jnp.full((_L,), e, jnp.int32)
            ev = jnp.exp(buf[e, pl.ds(t0, _L)] - m)
            s = s + ev
            gt1 = ev > v1
            gt2 = ev > v2
            v2n = jnp.where(gt1, v1, jnp.where(gt2, ev, v2))
            i2n = jnp.where(gt1, i1, jnp.where(gt2, ecol, i2))
            v1 = jnp.where(gt1, ev, v1)
            i1 = jnp.where(gt1, ecol, i1)
            v2, i2 = v2n, i2n
        tws[0, pl.ds(t0, _L)] = v1 / s
        tws[1, pl.ds(t0, _L)] = v2 / s
        tis[0, pl.ds(t0, _L)] = i1
        tis[1, pl.ds(t0, _L)] = i2
        return carry

    lax.fori_loop(0, _TPW // _L, group, 0)

    pltpu.sync_copy(tws, tw_hbm.at[:, pl.ds(base, _TPW)])
    pltpu.sync_copy(tis, ti_hbm.at[:, pl.ds(base, _TPW)])


_sc_top2 = functools.partial(
    pl.kernel,
    out_type=[
        jax.ShapeDtypeStruct((2, _TOKENS), jnp.float32),
        jax.ShapeDtypeStruct((2, _TOKENS), jnp.int32),
    ],
    mesh=plsc.VectorSubcoreMesh(
        core_axis_name="c", subcore_axis_name="s", num_cores=_NC,
        num_subcores=_NS),
    scratch_types=[
        pltpu.VMEM((_E, _TPW), jnp.float32),
        pltpu.VMEM((2, _TPW), jnp.float32),
        pltpu.VMEM((2, _TPW), jnp.int32),
    ],
)(_sc_top2_body)


@jax.jit
def _router(x, W, b):
    logits_t = _tc_logits_t(x, W, b)
    tw_t, ti_t = _sc_top2(logits_t)
    return tw_t.T, ti_t.T


def kernel(x, k, W, b):
    tw, ti = _router(x, W, b)
    ti = ti + (jnp.asarray(k, dtype=ti.dtype) - 2)
    return (tw, ti)
